# baseline (speedup 1.0000x reference)
import jax
import jax.numpy as jnp
from jax import lax
from jax.experimental import pallas as pl
from jax.experimental.pallas import tpu as pltpu

N_NODES = 100000
EPS = 1e-5
EFW = 18
GW = 8


def _tc_body(h_ref, ag_ref, afd_ref, normt_ref, wh_ref, wg_ref, wf_ref,
             b_ref, g_ref, beta_ref, o_ref):
    acc = jnp.dot(h_ref[...], wh_ref[...], preferred_element_type=jnp.float32)
    s = jnp.dot(ag_ref[...], wg_ref[...], preferred_element_type=jnp.float32)
    s += jnp.dot(afd_ref[...], wf_ref[...], preferred_element_type=jnp.float32)
    nrm = jnp.transpose(normt_ref[...])
    acc += s * nrm + b_ref[...]
    mu = jnp.mean(acc, axis=-1, keepdims=True)
    d = acc - mu
    var = jnp.mean(d * d, axis=-1, keepdims=True)
    y = d * lax.rsqrt(var + EPS) * g_ref[...] + beta_ref[...]
    o_ref[...] = jnp.maximum(y, 0.0)


def _dense_tail(h, ag, afd, norm, W, b, ln_gamma, ln_beta):
    out_feats, lin_in = W.shape
    in_feats = h.shape[1]
    wh = W[:, :in_feats].T
    wg = jnp.pad(W[:, in_feats:in_feats + 6].T, ((0, 2), (0, 0)))
    wf = W[:, in_feats + 6:].T
    b2 = b.reshape(1, -1)
    g2 = ln_gamma.reshape(1, -1)
    be2 = ln_beta.reshape(1, -1)
    normt = norm.T
    bn = 512
    grid = (pl.cdiv(N_NODES, bn),)
    return pl.pallas_call(
        _tc_body,
        grid=grid,
        in_specs=[
            pl.BlockSpec((bn, in_feats), lambda i: (i, 0)),
            pl.BlockSpec((bn, GW), lambda i: (i, 0)),
            pl.BlockSpec((bn, EFW), lambda i: (i, 0)),
            pl.BlockSpec((1, bn), lambda i: (0, i)),
            pl.BlockSpec((in_feats, out_feats), lambda i: (0, 0)),
            pl.BlockSpec((GW, out_feats), lambda i: (0, 0)),
            pl.BlockSpec((EFW, out_feats), lambda i: (0, 0)),
            pl.BlockSpec((1, out_feats), lambda i: (0, 0)),
            pl.BlockSpec((1, out_feats), lambda i: (0, 0)),
            pl.BlockSpec((1, out_feats), lambda i: (0, 0)),
        ],
        out_specs=pl.BlockSpec((bn, out_feats), lambda i: (i, 0)),
        out_shape=jax.ShapeDtypeStruct((N_NODES, out_feats), jnp.float32),
        compiler_params=pltpu.CompilerParams(
            dimension_semantics=("arbitrary",)),
    )(h, ag, afd, normt, wh, wg, wf, b2, g2, be2)


@jax.jit
def kernel(h, edge_index, geometric, distance, angle, feat,
           discrete_bin_edges, norm, W, b, ln_gamma, ln_beta):
    src = edge_index[0]
    dst = edge_index[1]
    geo8 = jnp.pad(geometric, ((0, 0), (0, 2)))
    fd = jnp.concatenate([distance[:, None], angle[:, None], feat,
                          discrete_bin_edges], axis=1)
    hsrc = jnp.take(geo8, src, axis=0)
    ag = jax.ops.segment_sum(hsrc, dst, num_segments=N_NODES)
    afd = jax.ops.segment_sum(fd, dst, num_segments=N_NODES)
    return _dense_tail(h, ag, afd, norm, W, b, ln_gamma, ln_beta)
